# baseline (speedup 1.0000x reference)
import numpy as np
import jax
import jax.numpy as jnp
from jax.experimental import pallas as pl
from jax.experimental.pallas import tpu as pltpu
from jax.experimental.pallas import tpu_sc as plsc

D_MODEL = 1024
LANES = 16


def _make_pe(seq_len: int) -> jnp.ndarray:
    position = np.arange(seq_len, dtype=np.float64)[:, None]
    div_term = np.exp(
        np.arange(0, D_MODEL, 2, dtype=np.float64) * (-np.log(10000.0) / D_MODEL)
    )
    pe = np.zeros((seq_len, D_MODEL), dtype=np.float32)
    pe[:, 0::2] = np.sin(position * div_term).astype(np.float32)
    pe[:, 1::2] = np.cos(position * div_term).astype(np.float32)
    return jnp.asarray(pe)


def _tc_body(xi_ref, xt_ref, pe_ref, emb_ref, oi_ref, ot_ref):
    pe = pe_ref[...]
    oi_ref[...] = xi_ref[...] + (pe + emb_ref[0, :])[None]
    ot_ref[...] = xt_ref[...] + (pe + emb_ref[1, :])[None]


def _tc_two(x_image, x_text, pe, emb2):
    B, S, D = x_image.shape
    bs = 512
    grid = (S // bs, B)
    x_spec = pl.BlockSpec((1, bs, D), lambda s, b: (b, s, 0))
    pe_spec = pl.BlockSpec((bs, D), lambda s, b: (s, 0))
    emb_spec = pl.BlockSpec((2, D), lambda s, b: (0, 0))
    out_shape = jax.ShapeDtypeStruct((B, S, D), x_image.dtype)
    return pl.pallas_call(
        _tc_body,
        grid=grid,
        in_specs=[x_spec, x_spec, pe_spec, emb_spec],
        out_specs=[x_spec, x_spec],
        out_shape=[out_shape, out_shape],
        compiler_params=pltpu.CompilerParams(
            dimension_semantics=("arbitrary", "arbitrary"),
        ),
    )(x_image, x_text, pe, emb2)


def _sc_one(x, pe, emb_row):
    B, S, D = x.shape
    bs, bd = 32, 128
    mesh = plsc.VectorSubcoreMesh(core_axis_name="core", subcore_axis_name="subcore")

    @pl.kernel(out_type=jax.ShapeDtypeStruct((B, S, D), x.dtype), mesh=mesh)
    def sc_kernel(x_hbm, pe_hbm, emb_hbm, o_hbm):
        def body(x_v, pe_v, emb_v, o_v):
            @pl.loop(0, bs)
            def _(r):
                @pl.loop(0, bd, step=LANES)
                def _(c):
                    pem = pe_v[r, pl.ds(c, LANES)] + emb_v[0, pl.ds(c, LANES)]
                    for b in range(B):
                        o_v[b, r, pl.ds(c, LANES)] = x_v[b, r, pl.ds(c, LANES)] + pem

        pltpu.emit_pipeline(
            body,
            grid=(S // bs, D // bd),
            in_specs=[
                pl.BlockSpec((B, bs, bd), lambda s, d: (0, s, d)),
                pl.BlockSpec((bs, bd), lambda s, d: (s, d)),
                pl.BlockSpec((1, bd), lambda s, d: (0, d)),
            ],
            out_specs=[pl.BlockSpec((B, bs, bd), lambda s, d: (0, s, d))],
            core_axis_name=("core", "subcore"),
            dimension_semantics=(pltpu.PARALLEL, pltpu.PARALLEL),
        )(x_hbm, pe_hbm, emb_hbm, o_hbm)

    return sc_kernel(x, pe, emb_row)


def kernel(x_sensor, x_image, x_text, emb_table):
    B, S, D = x_sensor.shape
    pe = _make_pe(S)
    out_sensor = _sc_one(x_sensor, pe, emb_table[0:1])
    out_image, out_text = _tc_two(x_image, x_text, pe, emb_table[1:3])
    return (out_sensor, out_image, out_text)


# speedup vs baseline: 1.3722x; 1.3722x over previous
import numpy as np
import jax
import jax.numpy as jnp
from jax.experimental import pallas as pl
from jax.experimental.pallas import tpu as pltpu
from jax.experimental.pallas import tpu_sc as plsc

D_MODEL = 1024
LANES = 16


def _make_pe(seq_len: int) -> jnp.ndarray:
    position = np.arange(seq_len, dtype=np.float64)[:, None]
    div_term = np.exp(
        np.arange(0, D_MODEL, 2, dtype=np.float64) * (-np.log(10000.0) / D_MODEL)
    )
    pe = np.zeros((seq_len, D_MODEL), dtype=np.float32)
    pe[:, 0::2] = np.sin(position * div_term).astype(np.float32)
    pe[:, 1::2] = np.cos(position * div_term).astype(np.float32)
    return jnp.asarray(pe)


def _tc_body(xi_ref, xt_ref, pe_ref, emb_ref, oi_ref, ot_ref):
    pe = pe_ref[...]
    oi_ref[...] = xi_ref[...] + (pe + emb_ref[0, :])[None]
    ot_ref[...] = xt_ref[...] + (pe + emb_ref[1, :])[None]


def _tc_two(x_image, x_text, pe, emb2):
    B, S, D = x_image.shape
    bs = 512
    grid = (S // bs, B)
    x_spec = pl.BlockSpec((1, bs, D), lambda s, b: (b, s, 0))
    pe_spec = pl.BlockSpec((bs, D), lambda s, b: (s, 0))
    emb_spec = pl.BlockSpec((2, D), lambda s, b: (0, 0))
    out_shape = jax.ShapeDtypeStruct((B, S, D), x_image.dtype)
    return pl.pallas_call(
        _tc_body,
        grid=grid,
        in_specs=[x_spec, x_spec, pe_spec, emb_spec],
        out_specs=[x_spec, x_spec],
        out_shape=[out_shape, out_shape],
        compiler_params=pltpu.CompilerParams(
            dimension_semantics=("arbitrary", "arbitrary"),
        ),
    )(x_image, x_text, pe, emb2)


def _sc_one(x, pe, emb_row):
    B, S, D = x.shape
    bs = 4
    mesh = plsc.VectorSubcoreMesh(core_axis_name="core", subcore_axis_name="subcore")

    @pl.kernel(out_type=jax.ShapeDtypeStruct((B, S, D), x.dtype), mesh=mesh)
    def sc_kernel(x_hbm, pe_hbm, emb_hbm, o_hbm):
        def body(x_v, pe_v, emb_v, o_v):
            @plsc.parallel_loop(0, D, step=LANES, unroll=2)
            def _(c):
                e = emb_v[0, pl.ds(c, LANES)]
                for r in range(bs):
                    pem = pe_v[r, pl.ds(c, LANES)] + e
                    for b in range(B):
                        o_v[b, r, pl.ds(c, LANES)] = x_v[b, r, pl.ds(c, LANES)] + pem

        pltpu.emit_pipeline(
            body,
            grid=(S // bs,),
            in_specs=[
                pl.BlockSpec((B, bs, D), lambda s: (0, s, 0)),
                pl.BlockSpec((bs, D), lambda s: (s, 0)),
                pl.BlockSpec((1, D), lambda s: (0, 0)),
            ],
            out_specs=[pl.BlockSpec((B, bs, D), lambda s: (0, s, 0))],
            core_axis_name=("core", "subcore"),
            dimension_semantics=(pltpu.PARALLEL,),
        )(x_hbm, pe_hbm, emb_hbm, o_hbm)

    return sc_kernel(x, pe, emb_row)


def kernel(x_sensor, x_image, x_text, emb_table):
    B, S, D = x_sensor.shape
    pe = _make_pe(S)
    out_sensor = _sc_one(x_sensor, pe, emb_table[0:1])
    out_image, out_text = _tc_two(x_image, x_text, pe, emb_table[1:3])
    return (out_sensor, out_image, out_text)


# speedup vs baseline: 1.3730x; 1.0006x over previous
import numpy as np
import jax
import jax.numpy as jnp
from jax.experimental import pallas as pl
from jax.experimental.pallas import tpu as pltpu
from jax.experimental.pallas import tpu_sc as plsc

D_MODEL = 1024
LANES = 16


def _make_pe_np(seq_len: int) -> np.ndarray:
    position = np.arange(seq_len, dtype=np.float64)[:, None]
    div_term = np.exp(
        np.arange(0, D_MODEL, 2, dtype=np.float64) * (-np.log(10000.0) / D_MODEL)
    )
    pe = np.zeros((seq_len, D_MODEL), dtype=np.float32)
    pe[:, 0::2] = np.sin(position * div_term).astype(np.float32)
    pe[:, 1::2] = np.cos(position * div_term).astype(np.float32)
    return pe


def _tc_body(xi_ref, xt_ref, pe_ref, emb_ref, oi_ref, ot_ref):
    pe = pe_ref[...]
    oi_ref[...] = xi_ref[...] + (pe + emb_ref[0, :])[None]
    ot_ref[...] = xt_ref[...] + (pe + emb_ref[1, :])[None]


def _tc_two(x_image, x_text, pe, emb2):
    B, S, D = x_image.shape
    bs = 512
    grid = (S // bs, B)
    x_spec = pl.BlockSpec((1, bs, D), lambda s, b: (b, s, 0))
    pe_spec = pl.BlockSpec((bs, D), lambda s, b: (s, 0))
    emb_spec = pl.BlockSpec((2, D), lambda s, b: (0, 0))
    out_shape = jax.ShapeDtypeStruct((B, S, D), x_image.dtype)
    return pl.pallas_call(
        _tc_body,
        grid=grid,
        in_specs=[x_spec, x_spec, pe_spec, emb_spec],
        out_specs=[x_spec, x_spec],
        out_shape=[out_shape, out_shape],
        compiler_params=pltpu.CompilerParams(
            dimension_semantics=("arbitrary", "arbitrary"),
        ),
    )(x_image, x_text, pe, emb2)


def _sc_one(x, pe3, emb_row):
    B, S, D = x.shape
    bs = 4
    mesh = plsc.VectorSubcoreMesh(core_axis_name="core", subcore_axis_name="subcore")

    @pl.kernel(out_type=jax.ShapeDtypeStruct((B, S, D), x.dtype), mesh=mesh)
    def sc_kernel(x_hbm, pe_hbm, emb_hbm, o_hbm):
        def body(x_v, pe_v, emb_v, o_v):
            @plsc.parallel_loop(0, D, step=LANES, unroll=4)
            def _(c):
                e = emb_v[0, pl.ds(c, LANES)]
                for r in range(bs):
                    pem = pe_v[0, r, pl.ds(c, LANES)] + e
                    for b in range(B):
                        o_v[b, r, pl.ds(c, LANES)] = x_v[b, r, pl.ds(c, LANES)] + pem

        pltpu.emit_pipeline(
            body,
            grid=(S // bs,),
            in_specs=[
                pl.BlockSpec((B, bs, D), lambda s: (0, s, 0)),
                pl.BlockSpec((1, bs, D), lambda s: (0, s, 0)),
                pl.BlockSpec((1, D), lambda s: (0, 0)),
            ],
            out_specs=[pl.BlockSpec((B, bs, D), lambda s: (0, s, 0))],
            core_axis_name=("core", "subcore"),
            dimension_semantics=(pltpu.PARALLEL,),
        )(x_hbm, pe_hbm, emb_hbm, o_hbm)

    return sc_kernel(x, pe3, emb_row)


def kernel(x_sensor, x_image, x_text, emb_table):
    B, S, D = x_sensor.shape
    pe_np = _make_pe_np(S)
    pe = jnp.asarray(pe_np)
    pe3 = jnp.asarray(pe_np[None])
    out_sensor = _sc_one(x_sensor, pe3, emb_table[0:1])
    out_image, out_text = _tc_two(x_image, x_text, pe, emb_table[1:3])
    return (out_sensor, out_image, out_text)
